# baseline (speedup 1.0000x reference)
import functools

import jax
import jax.numpy as jnp
from jax import lax
from jax.experimental import pallas as pl
from jax.experimental.pallas import tpu as pltpu
from jax.experimental.pallas import tpu_sc as plsc

_BATCH = 4
_ROWS = 8192
_DIM = 256
_NC = 2
_NS = 16
_CHUNK = _ROWS // (_NC * _NS)


def _sc_body(w_hbm, out_hbm, buf, sem):
    wid = lax.axis_index("s") * _NC + lax.axis_index("c")
    base = wid * _CHUNK
    pltpu.sync_copy(w_hbm.at[pl.ds(base, _CHUNK)], buf)
    copies = [
        pltpu.async_copy(buf, out_hbm.at[pl.ds(b * _ROWS + base, _CHUNK)], sem)
        for b in range(_BATCH)
    ]
    for c in copies:
        c.wait()


def kernel(tokens, W):
    del tokens
    mesh = plsc.VectorSubcoreMesh(core_axis_name="c", subcore_axis_name="s")
    run = functools.partial(
        pl.kernel,
        mesh=mesh,
        out_type=jax.ShapeDtypeStruct((_BATCH * _ROWS, _DIM), jnp.float32),
        scratch_types=[
            pltpu.VMEM((_CHUNK, _DIM), jnp.float32),
            pltpu.SemaphoreType.DMA,
        ],
    )(_sc_body)
    out2d = run(W)
    return out2d.reshape(_BATCH, _ROWS, _DIM)


# speedup vs baseline: 2.3842x; 2.3842x over previous
import jax
import jax.numpy as jnp
from jax import lax
from jax.experimental import pallas as pl
from jax.experimental.pallas import tpu as pltpu

_BATCH = 4
_ROWS = 8192
_DIM = 256
_BLOCK = 2048
_NB = _ROWS // _BLOCK


def _fanout_body(w_ref, out_ref, sems):
    i = pl.program_id(0)
    slot = lax.rem(i, 2)
    prev = lax.rem(i + 1, 2)

    def copies(s):
        return [
            pltpu.make_async_copy(
                w_ref,
                out_ref.at[b, pl.ds(i * _BLOCK, _BLOCK), :],
                sems.at[s, b],
            )
            for b in range(_BATCH)
        ]

    for c in copies(slot):
        c.start()

    @pl.when(i > 0)
    def _drain_prev():
        for c in copies(prev):
            c.wait()

    @pl.when(i == _NB - 1)
    def _drain_last():
        for c in copies(slot):
            c.wait()


def kernel(tokens, W):
    del tokens
    return pl.pallas_call(
        _fanout_body,
        grid=(_NB,),
        in_specs=[pl.BlockSpec((_BLOCK, _DIM), lambda i: (i, 0))],
        out_specs=pl.BlockSpec(memory_space=pl.ANY),
        out_shape=jax.ShapeDtypeStruct((_BATCH, _ROWS, _DIM), jnp.float32),
        scratch_shapes=[pltpu.SemaphoreType.DMA((2, _BATCH))],
        compiler_params=pltpu.CompilerParams(
            dimension_semantics=("arbitrary",),
        ),
    )(W)
